# baseline (speedup 1.0000x reference)
import functools

import jax
import jax.numpy as jnp
from jax import lax
from jax.experimental import pallas as pl
from jax.experimental.pallas import tpu as pltpu
from jax.experimental.pallas import tpu_sc as plsc

NUM_USERS = 10000
NUM_ITEMS = 40000
N = 50000
E = 800000
D = 64
N_LAYERS = 3
BETA = 0.001

NC = 2
NS = 16
HALF = N // NC
EPT = E // NS
C_MAIN = 96
N_MAIN = EPT // C_MAIN
C_TAIL = EPT - N_MAIN * C_MAIN
NRING = 4
DUMMY = HALF
STRIPE = 1564
SP_ROWS = 16 * STRIPE
FL = 1560
SUP = 8
SUPE = SUP * C_MAIN
NSUPER = N_MAIN // SUP


EPW = E // (NC * NS)
SUPB = 2272
NSUPB = 11
MAXCH = 200
STG = 160


def _bin_body(row_hbm, col_hbm, val_hbm, buck_hbm, cnt_hbm,
              brow, bcol, bval,
              st0c, st0i, st0v, st1c, st1i, st1v, cntv):
    c = lax.axis_index("c")
    s = lax.axis_index("s")
    w = s * NC + c
    ibase = w * EPW
    stages = ((st0c, st0i, st0v), (st1c, st1i, st1v))

    def append(st, cv, iv, vv, m, wh):
        plsc.store_compressed(st[0].at[pl.ds(wh, 16)], cv, mask=m)
        plsc.store_compressed(st[1].at[pl.ds(wh, 16)], iv, mask=m)
        plsc.store_compressed(st[2].at[pl.ds(wh, 16)], vv, mask=m)
        pc = plsc.all_reduce_population_count(m)
        return wh + pc[0]

    def flush_if_full(h, wh, fh):
        st = stages[h]

        def do_flush(args):
            whx, fhx = args
            for fld in range(3):
                pltpu.sync_copy(st[fld].at[pl.ds(0, 128)],
                                buck_hbm.at[h, w, fhx, fld])
                rem = st[fld][pl.ds(128, 16)]
                st[fld][pl.ds(0, 16)] = rem
            return whx - 128, fhx + 1

        return lax.cond(wh >= 128, do_flush, lambda a: a, (wh, fh))

    def group(rv, cv, vv, lm, carry):
        w0, w1, f0, f1 = carry
        m1 = rv >= HALF
        m0 = jnp.logical_not(m1)
        if lm is not None:
            m0 = jnp.logical_and(m0, lm)
            m1 = jnp.logical_and(m1, lm)
        vv_i = plsc.bitcast(vv, jnp.int32)
        w0 = append(stages[0], cv, rv, vv_i, m0, w0)
        w1 = append(stages[1], cv, rv - HALF, vv_i, m1, w1)
        w0, f0 = flush_if_full(0, w0, f0)
        w1, f1 = flush_if_full(1, w1, f1)
        return w0, w1, f0, f1

    def outer(k, carry):
        soff = pl.multiple_of(ibase + k * SUPB, 8)
        pltpu.sync_copy(row_hbm.at[pl.ds(soff, SUPB)], brow)
        pltpu.sync_copy(col_hbm.at[pl.ds(soff, SUPB)], bcol)
        pltpu.sync_copy(val_hbm.at[pl.ds(soff, SUPB)], bval)

        def inner(g, cr):
            go = g * 16
            return group(brow[pl.ds(go, 16)], bcol[pl.ds(go, 16)],
                         bval[pl.ds(go, 16)], None, cr)
        return lax.fori_loop(0, SUPB // 16, inner, carry)

    zero = jnp.int32(0)
    carry = lax.fori_loop(0, NSUPB, outer, (zero, zero, zero, zero))

    toff = pl.multiple_of(ibase + NSUPB * SUPB, 8)
    pltpu.sync_copy(row_hbm.at[pl.ds(toff, 8)], brow.at[pl.ds(0, 8)])
    pltpu.sync_copy(col_hbm.at[pl.ds(toff, 8)], bcol.at[pl.ds(0, 8)])
    pltpu.sync_copy(val_hbm.at[pl.ds(toff, 8)], bval.at[pl.ds(0, 8)])
    lm = lax.iota(jnp.int32, 16) < (EPW - NSUPB * SUPB)
    carry = group(brow[pl.ds(0, 16)], bcol[pl.ds(0, 16)],
                  bval[pl.ds(0, 16)], lm, carry)
    w0, w1, f0, f1 = carry

    nchs = []
    for h in range(2):
        st = stages[h]
        wh = (w0, w1)[h]
        fh = (f0, f1)[h]
        for blk in range(8):
            sl = pl.ds(blk * 16, 16)
            pos = lax.iota(jnp.int32, 16) + blk * 16
            keep = pos < wh
            st[0][sl] = jnp.where(keep, st[0][sl], 0)
            st[1][sl] = jnp.where(keep, st[1][sl], DUMMY)
            st[2][sl] = jnp.where(keep, st[2][sl], 0)
        for fld in range(3):
            pltpu.sync_copy(st[fld].at[pl.ds(0, 128)],
                            buck_hbm.at[h, w, fh, fld])
        nchs.append(fh + 1)

    lane = lax.iota(jnp.int32, 16)
    cnt16 = jnp.where(lane == 0, nchs[0], jnp.where(lane == 1, nchs[1], 0))
    cntv[pl.ds(0, 16)] = cnt16
    pltpu.sync_copy(cntv, cnt_hbm.at[w])


_bin = functools.partial(
    pl.kernel,
    out_type=(jax.ShapeDtypeStruct((2, NC * NS, MAXCH, 3, 128), jnp.int32),
              jax.ShapeDtypeStruct((NC * NS, 16), jnp.int32)),
    mesh=plsc.VectorSubcoreMesh(core_axis_name="c", subcore_axis_name="s"),
    compiler_params=pltpu.CompilerParams(use_tc_tiling_on_sc=False,
                                         needs_layout_passes=False),
    scratch_types=[
        pltpu.VMEM((SUPB,), jnp.int32),
        pltpu.VMEM((SUPB,), jnp.int32),
        pltpu.VMEM((SUPB,), jnp.float32),
        pltpu.VMEM((STG,), jnp.int32),
        pltpu.VMEM((STG,), jnp.int32),
        pltpu.VMEM((STG,), jnp.int32),
        pltpu.VMEM((STG,), jnp.int32),
        pltpu.VMEM((STG,), jnp.int32),
        pltpu.VMEM((STG,), jnp.int32),
        pltpu.VMEM((16,), jnp.int32),
    ],
)(_bin_body)


def _spmm_body(buck_hbm, cnt_hbm, emb_hbm, zer_hbm, out_hbm,
               acc, ebuf, rows, rowsf, idxb, cntv, esem, gsem, ssem):
    c = lax.axis_index("c")
    s = lax.axis_index("s")

    pltpu.sync_copy(zer_hbm, acc.at[pl.ds(s * STRIPE, STRIPE)])
    plsc.subcore_barrier()

    def drain_scatter():
        pltpu.make_async_copy(zer_hbm.at[pl.ds(0, 128)], rowsf.at[0],
                              ssem).wait()

    for wloc in range(2):
        w = 2 * s + wloc
        pltpu.sync_copy(cnt_hbm.at[w], cntv)
        cv16 = cntv[pl.ds(0, 16)]
        nch = jnp.where(c == 0, cv16[0], cv16[1])

        pltpu.sync_copy(buck_hbm.at[c, w, 0], ebuf.at[0])
        pltpu.async_copy(emb_hbm.at[ebuf.at[0, 0]], rows.at[0], gsem)

        @pl.when(nch > 1)
        def _pf1():
            pltpu.async_copy(buck_hbm.at[c, w, 1], ebuf.at[1], esem)

        def body(i, carry):
            p = lax.rem(i, 3)

            @pl.when(i >= 2)
            def _drain():
                drain_scatter()

            @pl.when(i < nch - 1)
            def _nxt():
                p1 = lax.rem(i + 1, 3)
                pltpu.make_async_copy(buck_hbm.at[c, w, 0], ebuf.at[0],
                                      esem).wait()
                pltpu.async_copy(emb_hbm.at[ebuf.at[p1, 0]], rows.at[p1],
                                 gsem)

                @pl.when(i < nch - 2)
                def _pf2():
                    pltpu.async_copy(buck_hbm.at[c, w, i + 2],
                                     ebuf.at[lax.rem(i + 2, 3)], esem)

            pp = lax.rem(i, 2)
            pltpu.make_async_copy(emb_hbm.at[pl.ds(0, 128)], rows.at[0],
                                  gsem).wait()
            for g in range(8):
                sl = pl.ds(g * 16, 16)
                idxb[p, sl] = ebuf[p, 1, sl]
                v16 = plsc.bitcast(ebuf[p, 2, sl], jnp.float32)
                for t in range(16):
                    e = g * 16 + t
                    v = v16[t]
                    for q in range(D // 32):
                        h32 = rows[p, e, pl.ds(q * 32, 32)]
                        a, b = plsc.unpack(
                            h32, format=plsc.PackFormat.INTERLEAVED)
                        rowsf[pp, e, pl.ds(q * 32, 16)] = a * v
                        rowsf[pp, e, pl.ds(q * 32 + 16, 16)] = b * v
            pltpu.async_copy(rowsf.at[pp], acc.at[idxb.at[p]], ssem,
                             add=True)
            return carry

        lax.fori_loop(0, nch, body, 0)

        @pl.when(nch >= 2)
        def _d2():
            drain_scatter()
        drain_scatter()

    plsc.subcore_barrier()
    fbase = s * FL
    pltpu.sync_copy(acc.at[pl.ds(fbase, FL)],
                    out_hbm.at[pl.ds(c * HALF + fbase, FL)])

    @pl.when(s == 0)
    def _flush_tail():
        pltpu.sync_copy(acc.at[pl.ds(NS * FL, HALF - NS * FL)],
                        out_hbm.at[pl.ds(c * HALF + NS * FL, HALF - NS * FL)])


_spmm = functools.partial(
    pl.kernel,
    out_type=jax.ShapeDtypeStruct((N, D), jnp.float32),
    mesh=plsc.VectorSubcoreMesh(core_axis_name="c", subcore_axis_name="s"),
    compiler_params=pltpu.CompilerParams(use_tc_tiling_on_sc=False,
                                         needs_layout_passes=False),
    scratch_types=[
        pltpu.VMEM_SHARED((SP_ROWS, D), jnp.float32),
        pltpu.VMEM((3, 3, 128), jnp.int32),
        pltpu.VMEM((3, 128, D), jnp.bfloat16),
        pltpu.VMEM((2, 128, D), jnp.float32),
        pltpu.VMEM((3, 128), jnp.int32),
        pltpu.VMEM((16,), jnp.int32),
        pltpu.SemaphoreType.DMA,
        pltpu.SemaphoreType.DMA,
        pltpu.SemaphoreType.DMA,
    ],
)(_spmm_body)


_PERM = []
for _q in range(D // 32):
    for _i in range(16):
        _PERM.extend((32 * _q + _i, 32 * _q + 16 + _i))
_PERM = tuple(_PERM)


def _spmm_call(buck, cnt, emb):
    zer = jnp.zeros((STRIPE, D), jnp.float32)
    embb = (emb.astype(jnp.bfloat16)
            .reshape(N, D // 32, 2, 16)
            .swapaxes(2, 3)
            .reshape(N, D))
    return _spmm(buck, cnt, embb, zer)


BN = 2000


def _dense_body(e_ref, h_ref, wih_ref, whh_ref, bih_ref, bhh_ref,
                wgc_ref, bgc_ref, o_ref):
    e = e_ref[...]
    h = h_ref[...]
    gi = jnp.dot(e, wih_ref[...], preferred_element_type=jnp.float32) + bih_ref[...]
    gh = jnp.dot(h, whh_ref[...], preferred_element_type=jnp.float32) + bhh_ref[...]
    i_r, i_z, i_n = gi[:, 0:D], gi[:, D:2 * D], gi[:, 2 * D:3 * D]
    h_r, h_z, h_n = gh[:, 0:D], gh[:, D:2 * D], gh[:, 2 * D:3 * D]
    r = jax.nn.sigmoid(i_r + h_r)
    z = jax.nn.sigmoid(i_z + h_z)
    n = jnp.tanh(i_n + r * h_n)
    gru = (1.0 - z) * n + z * h
    side = jnp.dot(e * gru, wgc_ref[...], preferred_element_type=jnp.float32)
    side = side + bgc_ref[...]
    x = side + e
    side = jnp.where(x >= 0.0, x, 0.2 * x)
    nrm = jnp.sqrt(jnp.sum(side * side, axis=1, keepdims=True))
    o_ref[...] = side / jnp.maximum(nrm, 1e-12)


def _dense_call(e, h, wihT, whhT, bih, bhh, wgc, bgc):
    return pl.pallas_call(
        _dense_body,
        grid=(N // BN,),
        in_specs=[
            pl.BlockSpec((BN, D), lambda i: (i, 0)),
            pl.BlockSpec((BN, D), lambda i: (i, 0)),
            pl.BlockSpec((D, 3 * D), lambda i: (0, 0)),
            pl.BlockSpec((D, 3 * D), lambda i: (0, 0)),
            pl.BlockSpec((1, 3 * D), lambda i: (0, 0)),
            pl.BlockSpec((1, 3 * D), lambda i: (0, 0)),
            pl.BlockSpec((D, D), lambda i: (0, 0)),
            pl.BlockSpec((1, D), lambda i: (0, 0)),
        ],
        out_specs=pl.BlockSpec((BN, D), lambda i: (i, 0)),
        out_shape=jax.ShapeDtypeStruct((N, D), jnp.float32),
    )(e, h, wihT, whhT, bih, bhh, wgc, bgc)


BI = 1024
NU = 1024
NSTEP = 40
TAIL = NUM_ITEMS - (NSTEP - 1) * BI


def _final_body(u_ref, e0, e1, e2, e3, e4, e5, e6, e7, o_hbm,
                itb, itt, obuf, obt, isem, osem):
    i = pl.program_id(0)
    srcs = (e0, e1, e2, e3, e4, e5, e6, e7)

    @pl.when(i < NSTEP - 1)
    def _main():
        base = NUM_USERS + i * BI
        for a in range(8):
            pltpu.sync_copy(srcs[a].at[pl.ds(base, BI)], itb.at[a])
        it = (itb[0] + itb[1] + itb[2] + itb[3]) * 0.25 \
            + BETA * (itb[4] + itb[5] + itb[6] + itb[7])
        acc = lax.dot_general(u_ref[...], it, (((1,), (1,)), ((), ())),
                              preferred_element_type=jnp.float32)
        obuf[...] = jax.nn.sigmoid(acc)
        pltpu.async_copy(obuf, o_hbm.at[:, pl.ds(i * BI, BI)], osem).wait()

    @pl.when(i == NSTEP - 1)
    def _tail():
        base = NUM_USERS + (NSTEP - 1) * BI
        for a in range(8):
            pltpu.sync_copy(srcs[a].at[pl.ds(base, TAIL)], itt.at[a])
        it = (itt[0] + itt[1] + itt[2] + itt[3]) * 0.25 \
            + BETA * (itt[4] + itt[5] + itt[6] + itt[7])
        acc = lax.dot_general(u_ref[...], it, (((1,), (1,)), ((), ())),
                              preferred_element_type=jnp.float32)
        obt[...] = jax.nn.sigmoid(acc)
        pltpu.async_copy(obt, o_hbm.at[:, pl.ds((NSTEP - 1) * BI, TAIL)],
                         osem).wait()


def _final_call(u, arrs1, arrs2):
    hbm_spec = pl.BlockSpec(memory_space=pltpu.MemorySpace.HBM)
    return pl.pallas_call(
        _final_body,
        grid=(NSTEP,),
        in_specs=[pl.BlockSpec((NU, D), lambda i: (0, 0))] + [hbm_spec] * 8,
        out_specs=hbm_spec,
        out_shape=jax.ShapeDtypeStruct((NU, NUM_ITEMS), jnp.float32),
        scratch_shapes=[pltpu.VMEM((8, BI, D), jnp.float32),
                        pltpu.VMEM((8, TAIL, D), jnp.float32),
                        pltpu.VMEM((NU, BI), jnp.float32),
                        pltpu.VMEM((NU, TAIL), jnp.float32),
                        pltpu.SemaphoreType.DMA,
                        pltpu.SemaphoreType.DMA],
    )(u, *arrs1, *arrs2)


def kernel(users, edge_index, edge_values, user_table, item_table,
           w_ih, w_hh, b_ih, b_hh, W_gc, b_gc, h0):
    row = edge_index[0]
    col = edge_index[1]
    all_emb = jnp.concatenate([user_table, item_table], axis=0)
    buck, cnt = _bin(row, col, edge_values)

    wihT = w_ih.T
    whhT = w_hh.T
    bih = b_ih.reshape(1, 3 * D)
    bhh = b_hh.reshape(1, 3 * D)

    e10 = all_emb
    s0 = _dense_call(all_emb, h0[0, 0], wihT, whhT, bih, bhh,
                     W_gc[0], b_gc[0])
    e11 = _spmm_call(buck, cnt, e10)
    e21 = _spmm_call(buck, cnt, s0)
    s1 = _dense_call(e21, h0[1, 0], wihT, whhT, bih, bhh, W_gc[1], b_gc[1])
    e12 = _spmm_call(buck, cnt, e11)
    e22 = _spmm_call(buck, cnt, s1)
    s2 = _dense_call(e22, h0[2, 0], wihT, whhT, bih, bhh, W_gc[2], b_gc[2])
    e13 = _spmm_call(buck, cnt, e12)
    e23 = _spmm_call(buck, cnt, s2)

    u1 = (e10[users] + e11[users] + e12[users] + e13[users]) * 0.25
    u2 = all_emb[users] + e21[users] + e22[users] + e23[users]
    u = u1 + BETA * u2
    return _final_call(u, (e10, e11, e12, e13), (all_emb, e21, e22, e23))


# speedup vs baseline: 1.9833x; 1.9833x over previous
import functools

import jax
import jax.numpy as jnp
from jax import lax
from jax.experimental import pallas as pl
from jax.experimental.pallas import tpu as pltpu
from jax.experimental.pallas import tpu_sc as plsc

NUM_USERS = 10000
NUM_ITEMS = 40000
N = 50000
E = 800000
D = 64
N_LAYERS = 3
BETA = 0.001

NC = 2
NS = 16
HALF = N // NC
EPT = E // NS
C_MAIN = 96
N_MAIN = EPT // C_MAIN
C_TAIL = EPT - N_MAIN * C_MAIN
NRING = 4
DUMMY = HALF
STRIPE = 1564
SP_ROWS = 16 * STRIPE
FL = 1560
SUP = 8
SUPE = SUP * C_MAIN
NSUPER = N_MAIN // SUP


EPW = E // (NC * NS)
SUPB = 2272
NSUPB = 11
MAXCH = 200
STG = 160


def _bin_body(row_hbm, col_hbm, val_hbm, buck_hbm, cnt_hbm,
              brow, bcol, bval,
              st0c, st0i, st0v, st1c, st1i, st1v, cntv):
    c = lax.axis_index("c")
    s = lax.axis_index("s")
    w = s * NC + c
    ibase = w * EPW
    stages = ((st0c, st0i, st0v), (st1c, st1i, st1v))

    def append(st, cv, iv, vv, m, wh):
        plsc.store_compressed(st[0].at[pl.ds(wh, 16)], cv, mask=m)
        plsc.store_compressed(st[1].at[pl.ds(wh, 16)], iv, mask=m)
        plsc.store_compressed(st[2].at[pl.ds(wh, 16)], vv, mask=m)
        pc = plsc.all_reduce_population_count(m)
        return wh + pc[0]

    def flush_if_full(h, wh, fh):
        st = stages[h]

        def do_flush(args):
            whx, fhx = args
            for fld in range(3):
                pltpu.sync_copy(st[fld].at[pl.ds(0, 128)],
                                buck_hbm.at[h, w, fhx, fld])
                rem = st[fld][pl.ds(128, 16)]
                st[fld][pl.ds(0, 16)] = rem
            return whx - 128, fhx + 1

        return lax.cond(wh >= 128, do_flush, lambda a: a, (wh, fh))

    def group(rv, cv, vv, lm, carry):
        w0, w1, f0, f1 = carry
        m1 = rv >= HALF
        m0 = jnp.logical_not(m1)
        if lm is not None:
            m0 = jnp.logical_and(m0, lm)
            m1 = jnp.logical_and(m1, lm)
        vv_i = plsc.bitcast(vv, jnp.int32)
        w0 = append(stages[0], cv, rv, vv_i, m0, w0)
        w1 = append(stages[1], cv, rv - HALF, vv_i, m1, w1)
        w0, f0 = flush_if_full(0, w0, f0)
        w1, f1 = flush_if_full(1, w1, f1)
        return w0, w1, f0, f1

    def outer(k, carry):
        soff = pl.multiple_of(ibase + k * SUPB, 8)
        pltpu.sync_copy(row_hbm.at[pl.ds(soff, SUPB)], brow)
        pltpu.sync_copy(col_hbm.at[pl.ds(soff, SUPB)], bcol)
        pltpu.sync_copy(val_hbm.at[pl.ds(soff, SUPB)], bval)

        def inner(g, cr):
            go = g * 16
            return group(brow[pl.ds(go, 16)], bcol[pl.ds(go, 16)],
                         bval[pl.ds(go, 16)], None, cr)
        return lax.fori_loop(0, SUPB // 16, inner, carry)

    zero = jnp.int32(0)
    carry = lax.fori_loop(0, NSUPB, outer, (zero, zero, zero, zero))

    toff = pl.multiple_of(ibase + NSUPB * SUPB, 8)
    pltpu.sync_copy(row_hbm.at[pl.ds(toff, 8)], brow.at[pl.ds(0, 8)])
    pltpu.sync_copy(col_hbm.at[pl.ds(toff, 8)], bcol.at[pl.ds(0, 8)])
    pltpu.sync_copy(val_hbm.at[pl.ds(toff, 8)], bval.at[pl.ds(0, 8)])
    lm = lax.iota(jnp.int32, 16) < (EPW - NSUPB * SUPB)
    carry = group(brow[pl.ds(0, 16)], bcol[pl.ds(0, 16)],
                  bval[pl.ds(0, 16)], lm, carry)
    w0, w1, f0, f1 = carry

    nchs = []
    for h in range(2):
        st = stages[h]
        wh = (w0, w1)[h]
        fh = (f0, f1)[h]
        for blk in range(8):
            sl = pl.ds(blk * 16, 16)
            pos = lax.iota(jnp.int32, 16) + blk * 16
            keep = pos < wh
            st[0][sl] = jnp.where(keep, st[0][sl], 0)
            st[1][sl] = jnp.where(keep, st[1][sl], DUMMY)
            st[2][sl] = jnp.where(keep, st[2][sl], 0)
        for fld in range(3):
            pltpu.sync_copy(st[fld].at[pl.ds(0, 128)],
                            buck_hbm.at[h, w, fh, fld])
        nchs.append(fh + 1)

    lane = lax.iota(jnp.int32, 16)
    cnt16 = jnp.where(lane == 0, nchs[0], jnp.where(lane == 1, nchs[1], 0))
    cntv[pl.ds(0, 16)] = cnt16
    pltpu.sync_copy(cntv, cnt_hbm.at[w])


_bin = functools.partial(
    pl.kernel,
    out_type=(jax.ShapeDtypeStruct((2, NC * NS, MAXCH, 3, 128), jnp.int32),
              jax.ShapeDtypeStruct((NC * NS, 16), jnp.int32)),
    mesh=plsc.VectorSubcoreMesh(core_axis_name="c", subcore_axis_name="s"),
    compiler_params=pltpu.CompilerParams(use_tc_tiling_on_sc=False,
                                         needs_layout_passes=False),
    scratch_types=[
        pltpu.VMEM((SUPB,), jnp.int32),
        pltpu.VMEM((SUPB,), jnp.int32),
        pltpu.VMEM((SUPB,), jnp.float32),
        pltpu.VMEM((STG,), jnp.int32),
        pltpu.VMEM((STG,), jnp.int32),
        pltpu.VMEM((STG,), jnp.int32),
        pltpu.VMEM((STG,), jnp.int32),
        pltpu.VMEM((STG,), jnp.int32),
        pltpu.VMEM((STG,), jnp.int32),
        pltpu.VMEM((16,), jnp.int32),
    ],
)(_bin_body)


def _spmm_body(buck_hbm, cnt_hbm, emb_hbm, zer_hbm, out_hbm,
               acc, ebuf, rows, idxb, cntv, esem, gsem, ssem):
    c = lax.axis_index("c")
    s = lax.axis_index("s")

    pltpu.sync_copy(zer_hbm, acc.at[pl.ds(s * STRIPE, STRIPE)])
    plsc.subcore_barrier()

    def drain_scatter():
        pltpu.make_async_copy(zer_hbm.at[pl.ds(0, 128)], rows.at[0],
                              ssem).wait()

    for wloc in range(2):
        w = 2 * s + wloc
        pltpu.sync_copy(cnt_hbm.at[w], cntv)
        cv16 = cntv[pl.ds(0, 16)]
        nch = jnp.where(c == 0, cv16[0], cv16[1])

        pltpu.sync_copy(buck_hbm.at[c, w, 0], ebuf.at[0])
        pltpu.async_copy(emb_hbm.at[ebuf.at[0, 0]], rows.at[0], gsem)

        @pl.when(nch > 1)
        def _pf1():
            pltpu.async_copy(buck_hbm.at[c, w, 1], ebuf.at[1], esem)

        def body(i, carry):
            p = lax.rem(i, 3)

            @pl.when(i >= 2)
            def _drain():
                drain_scatter()

            @pl.when(i < nch - 1)
            def _nxt():
                p1 = lax.rem(i + 1, 3)
                pltpu.make_async_copy(buck_hbm.at[c, w, 0], ebuf.at[0],
                                      esem).wait()
                pltpu.async_copy(emb_hbm.at[ebuf.at[p1, 0]], rows.at[p1],
                                 gsem)

                @pl.when(i < nch - 2)
                def _pf2():
                    pltpu.async_copy(buck_hbm.at[c, w, i + 2],
                                     ebuf.at[lax.rem(i + 2, 3)], esem)

            pltpu.make_async_copy(zer_hbm.at[pl.ds(0, 128)], rows.at[0],
                                  gsem).wait()
            for g in range(8):
                sl = pl.ds(g * 16, 16)
                idxb[p, sl] = ebuf[p, 1, sl]
                v16 = plsc.bitcast(ebuf[p, 2, sl], jnp.float32)
                for t in range(16):
                    e = g * 16 + t
                    v = v16[t]
                    for q in range(D // 16):
                        qs = pl.ds(q * 16, 16)
                        rows[p, e, qs] = rows[p, e, qs] * v
            pltpu.async_copy(rows.at[p], acc.at[idxb.at[p]], ssem, add=True)
            return carry

        lax.fori_loop(0, nch, body, 0)

        @pl.when(nch >= 2)
        def _d2():
            drain_scatter()
        drain_scatter()

    plsc.subcore_barrier()
    fbase = s * FL
    pltpu.sync_copy(acc.at[pl.ds(fbase, FL)],
                    out_hbm.at[pl.ds(c * HALF + fbase, FL)])

    @pl.when(s == 0)
    def _flush_tail():
        pltpu.sync_copy(acc.at[pl.ds(NS * FL, HALF - NS * FL)],
                        out_hbm.at[pl.ds(c * HALF + NS * FL, HALF - NS * FL)])


_spmm = functools.partial(
    pl.kernel,
    out_type=jax.ShapeDtypeStruct((N, D), jnp.float32),
    mesh=plsc.VectorSubcoreMesh(core_axis_name="c", subcore_axis_name="s"),
    compiler_params=pltpu.CompilerParams(use_tc_tiling_on_sc=False,
                                         needs_layout_passes=False),
    scratch_types=[
        pltpu.VMEM_SHARED((SP_ROWS, D), jnp.float32),
        pltpu.VMEM((3, 3, 128), jnp.int32),
        pltpu.VMEM((3, 128, D), jnp.float32),
        pltpu.VMEM((3, 128), jnp.int32),
        pltpu.VMEM((16,), jnp.int32),
        pltpu.SemaphoreType.DMA,
        pltpu.SemaphoreType.DMA,
        pltpu.SemaphoreType.DMA,
    ],
)(_spmm_body)


def _spmm_call(buck, cnt, emb):
    zer = jnp.zeros((STRIPE, D), jnp.float32)
    return _spmm(buck, cnt, emb, zer)


BN = 2000


def _dense_body(e_ref, h_ref, wih_ref, whh_ref, bih_ref, bhh_ref,
                wgc_ref, bgc_ref, o_ref):
    e = e_ref[...]
    h = h_ref[...]
    gi = jnp.dot(e, wih_ref[...], preferred_element_type=jnp.float32) + bih_ref[...]
    gh = jnp.dot(h, whh_ref[...], preferred_element_type=jnp.float32) + bhh_ref[...]
    i_r, i_z, i_n = gi[:, 0:D], gi[:, D:2 * D], gi[:, 2 * D:3 * D]
    h_r, h_z, h_n = gh[:, 0:D], gh[:, D:2 * D], gh[:, 2 * D:3 * D]
    r = jax.nn.sigmoid(i_r + h_r)
    z = jax.nn.sigmoid(i_z + h_z)
    n = jnp.tanh(i_n + r * h_n)
    gru = (1.0 - z) * n + z * h
    side = jnp.dot(e * gru, wgc_ref[...], preferred_element_type=jnp.float32)
    side = side + bgc_ref[...]
    x = side + e
    side = jnp.where(x >= 0.0, x, 0.2 * x)
    nrm = jnp.sqrt(jnp.sum(side * side, axis=1, keepdims=True))
    o_ref[...] = side / jnp.maximum(nrm, 1e-12)


def _dense_call(e, h, wihT, whhT, bih, bhh, wgc, bgc):
    return pl.pallas_call(
        _dense_body,
        grid=(N // BN,),
        in_specs=[
            pl.BlockSpec((BN, D), lambda i: (i, 0)),
            pl.BlockSpec((BN, D), lambda i: (i, 0)),
            pl.BlockSpec((D, 3 * D), lambda i: (0, 0)),
            pl.BlockSpec((D, 3 * D), lambda i: (0, 0)),
            pl.BlockSpec((1, 3 * D), lambda i: (0, 0)),
            pl.BlockSpec((1, 3 * D), lambda i: (0, 0)),
            pl.BlockSpec((D, D), lambda i: (0, 0)),
            pl.BlockSpec((1, D), lambda i: (0, 0)),
        ],
        out_specs=pl.BlockSpec((BN, D), lambda i: (i, 0)),
        out_shape=jax.ShapeDtypeStruct((N, D), jnp.float32),
    )(e, h, wihT, whhT, bih, bhh, wgc, bgc)


BI = 1024
NU = 1024
NSTEP = 40
TAIL = NUM_ITEMS - (NSTEP - 1) * BI


def _final_body(u_ref, e0, e1, e2, e3, e4, e5, e6, e7, o_hbm,
                itb, itt, oba, obb, obt, isem, osem):
    i = pl.program_id(0)
    srcs = (e0, e1, e2, e3, e4, e5, e6, e7)

    def drain_prev():
        pltpu.make_async_copy(oba, o_hbm.at[:, pl.ds(0, BI)], osem).wait()

    @pl.when(i < NSTEP - 1)
    def _main():
        base = NUM_USERS + i * BI
        for a in range(8):
            pltpu.sync_copy(srcs[a].at[pl.ds(base, BI)], itb.at[a])
        it = (itb[0] + itb[1] + itb[2] + itb[3]) * 0.25 \
            + BETA * (itb[4] + itb[5] + itb[6] + itb[7])
        acc = lax.dot_general(u_ref[...], it, (((1,), (1,)), ((), ())),
                              preferred_element_type=jnp.float32)
        res = jax.nn.sigmoid(acc)

        @pl.when(i > 0)
        def _dr():
            drain_prev()

        @pl.when(lax.rem(i, 2) == 0)
        def _even():
            oba[...] = res
            pltpu.async_copy(oba, o_hbm.at[:, pl.ds(i * BI, BI)], osem)

        @pl.when(lax.rem(i, 2) == 1)
        def _odd():
            obb[...] = res
            pltpu.async_copy(obb, o_hbm.at[:, pl.ds(i * BI, BI)], osem)

    @pl.when(i == NSTEP - 1)
    def _tail():
        base = NUM_USERS + (NSTEP - 1) * BI
        for a in range(8):
            pltpu.sync_copy(srcs[a].at[pl.ds(base, TAIL)], itt.at[a])
        it = (itt[0] + itt[1] + itt[2] + itt[3]) * 0.25 \
            + BETA * (itt[4] + itt[5] + itt[6] + itt[7])
        acc = lax.dot_general(u_ref[...], it, (((1,), (1,)), ((), ())),
                              preferred_element_type=jnp.float32)
        obt[...] = jax.nn.sigmoid(acc)
        drain_prev()
        pltpu.async_copy(obt, o_hbm.at[:, pl.ds((NSTEP - 1) * BI, TAIL)],
                         osem).wait()


def _final_call(u, arrs1, arrs2):
    hbm_spec = pl.BlockSpec(memory_space=pltpu.MemorySpace.HBM)
    return pl.pallas_call(
        _final_body,
        grid=(NSTEP,),
        in_specs=[pl.BlockSpec((NU, D), lambda i: (0, 0))] + [hbm_spec] * 8,
        out_specs=hbm_spec,
        out_shape=jax.ShapeDtypeStruct((NU, NUM_ITEMS), jnp.float32),
        scratch_shapes=[pltpu.VMEM((8, BI, D), jnp.float32),
                        pltpu.VMEM((8, TAIL, D), jnp.float32),
                        pltpu.VMEM((NU, BI), jnp.float32),
                        pltpu.VMEM((NU, BI), jnp.float32),
                        pltpu.VMEM((NU, TAIL), jnp.float32),
                        pltpu.SemaphoreType.DMA,
                        pltpu.SemaphoreType.DMA],
    )(u, *arrs1, *arrs2)


def kernel(users, edge_index, edge_values, user_table, item_table,
           w_ih, w_hh, b_ih, b_hh, W_gc, b_gc, h0):
    row = edge_index[0]
    col = edge_index[1]
    all_emb = jnp.concatenate([user_table, item_table], axis=0)
    buck, cnt = _bin(row, col, edge_values)

    wihT = w_ih.T
    whhT = w_hh.T
    bih = b_ih.reshape(1, 3 * D)
    bhh = b_hh.reshape(1, 3 * D)

    e10 = all_emb
    s0 = _dense_call(all_emb, h0[0, 0], wihT, whhT, bih, bhh,
                     W_gc[0], b_gc[0])
    e11 = _spmm_call(buck, cnt, e10)
    e21 = _spmm_call(buck, cnt, s0)
    s1 = _dense_call(e21, h0[1, 0], wihT, whhT, bih, bhh, W_gc[1], b_gc[1])
    e12 = _spmm_call(buck, cnt, e11)
    e22 = _spmm_call(buck, cnt, s1)
    s2 = _dense_call(e22, h0[2, 0], wihT, whhT, bih, bhh, W_gc[2], b_gc[2])
    e13 = _spmm_call(buck, cnt, e12)
    e23 = _spmm_call(buck, cnt, s2)

    u1 = (e10[users] + e11[users] + e12[users] + e13[users]) * 0.25
    u2 = all_emb[users] + e21[users] + e22[users] + e23[users]
    u = u1 + BETA * u2
    return _final_call(u, (e10, e11, e12, e13), (all_emb, e21, e22, e23))
